# baseline (speedup 1.0000x reference)
import functools

import jax
import jax.numpy as jnp
from jax import lax
from jax.experimental import pallas as pl
from jax.experimental.pallas import tpu as pltpu
from jax.experimental.pallas import tpu_sc as plsc

NC = 2
NS = 16
NW = NC * NS
CH = 128
LANES = 16


def _mm_body(x_ref, t_ref, o0_ref, o1_ref, o2_ref):
    y = lax.dot_general(x_ref[...], t_ref[...], (((0,), (0,)), ((), ())),
                        preferred_element_type=jnp.float32)
    c = o0_ref.shape[1]
    o0_ref[...] = y[:, 0:c]
    o1_ref[...] = y[:, c:2 * c]
    o2_ref[...] = y[:, 2 * c:3 * c]


def _mm3(x2d, thetaT):
    cin, m = x2d.shape
    ck3 = thetaT.shape[1]
    c = ck3 // 3
    out = jax.ShapeDtypeStruct((m, c), jnp.float32)
    return pl.pallas_call(
        _mm_body,
        grid=(1,),
        in_specs=[
            pl.BlockSpec((cin, m), lambda i: (0, 0)),
            pl.BlockSpec((cin, ck3), lambda i: (0, 0)),
        ],
        out_specs=[pl.BlockSpec((m, c), lambda i: (0, 0))] * 3,
        out_shape=[out, out, out],
    )(x2d, thetaT)


def _add2_body(a_ref, b_ref, d_ref, o_ref):
    o_ref[...] = a_ref[...] + b_ref[...] + d_ref[...]


def _add2(a, b, brow, bm=2000):
    m, ch = a.shape
    spec = pl.BlockSpec((bm, ch), lambda i: (i, 0))
    return pl.pallas_call(
        _add2_body,
        grid=(m // bm,),
        in_specs=[spec, spec, pl.BlockSpec((1, ch), lambda i: (0, 0))],
        out_specs=spec,
        out_shape=jax.ShapeDtypeStruct((m, ch), jnp.float32),
    )(a, b, brow)


def _vgather(vec, idx16):
    dnums = lax.GatherDimensionNumbers(
        offset_dims=(), collapsed_slice_dims=(0,), start_index_map=(0,))
    return lax.gather(vec, idx16[:, None], dnums, (1,),
                      mode=lax.GatherScatterMode.PROMISE_IN_BOUNDS)


def _spmm_partials(rows, cols, vals, table, init):
    nnzp = vals.shape[0]
    m, c = table.shape
    nt = nnzp // (NW * CH)
    rpt = 8 * (m // 8 // NS)
    rem = m - NS * rpt

    mesh = plsc.VectorSubcoreMesh(core_axis_name="c", subcore_axis_name="s")

    scratch = [
        pltpu.VMEM((CH, c), jnp.float32),
        pltpu.VMEM((CH,), jnp.int32),
        pltpu.VMEM((CH,), jnp.int32),
        pltpu.VMEM((CH,), jnp.float32),
        pltpu.VMEM_SHARED((m, c), jnp.float32),
        pltpu.SemaphoreType.DMA,
    ]

    @functools.partial(
        pl.kernel,
        out_type=jax.ShapeDtypeStruct((NC, m, c), jnp.float32),
        mesh=mesh,
        scratch_types=scratch,
    )
    def spmm(rows_hbm, cols_hbm, vals_hbm, table_hbm, init_hbm, out_hbm,
             gath, rowv, colv, valv, acc, sem):
        cid = lax.axis_index("c")
        sid = lax.axis_index("s")
        wid = cid * NS + sid
        ebase = wid * nt * CH

        pltpu.sync_copy(init_hbm.at[cid, pl.ds(sid * rpt, rpt)],
                        acc.at[pl.ds(sid * rpt, rpt)])
        if rem:
            @pl.when(sid == NS - 1)
            def _():
                pltpu.sync_copy(init_hbm.at[cid, pl.ds(NS * rpt, rem)],
                                acc.at[pl.ds(NS * rpt, rem)])
        plsc.subcore_barrier()

        def chunk_body(t, carry):
            e0 = ebase + t * CH
            pltpu.sync_copy(rows_hbm.at[pl.ds(e0, CH)], rowv)
            pltpu.sync_copy(cols_hbm.at[pl.ds(e0, CH)], colv)
            pltpu.sync_copy(vals_hbm.at[pl.ds(e0, CH)], valv)
            pltpu.async_copy(table_hbm.at[colv], gath, sem).wait()

            def scale_block(eb, cc):
                vblock = valv[pl.ds(eb * LANES, LANES)]
                for l in range(LANES):
                    vv = _vgather(vblock, jnp.full((LANES,), l, jnp.int32))
                    e = eb * LANES + l
                    for j in range(c // LANES):
                        g = gath[e, pl.ds(j * LANES, LANES)]
                        gath[e, pl.ds(j * LANES, LANES)] = g * vv
                return cc

            lax.fori_loop(0, CH // LANES, scale_block, 0)
            pltpu.sync_copy(gath, acc.at[rowv], add=True)
            return carry

        lax.fori_loop(0, nt, chunk_body, 0)
        plsc.subcore_barrier()

        pltpu.sync_copy(acc.at[pl.ds(sid * rpt, rpt)],
                        out_hbm.at[cid, pl.ds(sid * rpt, rpt)])
        if rem:
            @pl.when(sid == NS - 1)
            def _():
                pltpu.sync_copy(acc.at[pl.ds(NS * rpt, rem)],
                                out_hbm.at[cid, pl.ds(NS * rpt, rem)])

    return spmm(rows, cols, vals, table, init)


def kernel(L_indices, L_values, x, theta, bias):
    rows = L_indices[0].astype(jnp.int32)
    cols = L_indices[1].astype(jnp.int32)
    vals = L_values.astype(jnp.float32)

    cout, cin, k = theta.shape
    nnz = vals.shape[0]

    quant = NW * CH
    nnzp = ((nnz + quant - 1) // quant) * quant
    pad = nnzp - nnz
    if pad:
        rows = jnp.concatenate([rows, jnp.zeros((pad,), jnp.int32)])
        cols = jnp.concatenate([cols, jnp.zeros((pad,), jnp.int32)])
        vals = jnp.concatenate([vals, jnp.zeros((pad,), jnp.float32)])

    thetaT = jnp.transpose(theta, (1, 2, 0)).reshape(cin, k * cout)
    biasT = bias[0, :, 0][None, :]
    zrow = jnp.zeros((1, cout), jnp.float32)
    zeros_mc = jnp.zeros((x.shape[2], cout), jnp.float32)

    z0, z1, z2 = _mm3(x[0], thetaT)

    u_p = _spmm_partials(rows, cols, vals, z2, jnp.stack([z1, zeros_mc]))
    u = _add2(u_p[0], u_p[1], zrow)
    y_p = _spmm_partials(rows, cols, vals, u, jnp.stack([z0, zeros_mc]))
    yT = _add2(y_p[0], y_p[1], biasT)
    return yT.T[None]


# speedup vs baseline: 1.0543x; 1.0543x over previous
import functools

import jax
import jax.numpy as jnp
from jax import lax
from jax.experimental import pallas as pl
from jax.experimental.pallas import tpu as pltpu
from jax.experimental.pallas import tpu_sc as plsc

NC = 2
NS = 16
NW = NC * NS
CH = 128
LANES = 16


def _mm_body(x_ref, t_ref, o0_ref, o1_ref, o2_ref):
    y = lax.dot_general(x_ref[...], t_ref[...], (((0,), (0,)), ((), ())),
                        preferred_element_type=jnp.float32)
    c = o0_ref.shape[1]
    o0_ref[...] = y[:, 0:c]
    o1_ref[...] = y[:, c:2 * c]
    o2_ref[...] = y[:, 2 * c:3 * c]


def _mm3(x2d, thetaT):
    cin, m = x2d.shape
    ck3 = thetaT.shape[1]
    c = ck3 // 3
    out = jax.ShapeDtypeStruct((m, c), jnp.float32)
    return pl.pallas_call(
        _mm_body,
        grid=(1,),
        in_specs=[
            pl.BlockSpec((cin, m), lambda i: (0, 0)),
            pl.BlockSpec((cin, ck3), lambda i: (0, 0)),
        ],
        out_specs=[pl.BlockSpec((m, c), lambda i: (0, 0))] * 3,
        out_shape=[out, out, out],
    )(x2d, thetaT)


def _add2_body(a_ref, b_ref, d_ref, o_ref):
    o_ref[...] = a_ref[...] + b_ref[...] + d_ref[...]


def _add2(a, b, brow, bm=2000):
    m, ch = a.shape
    spec = pl.BlockSpec((bm, ch), lambda i: (i, 0))
    return pl.pallas_call(
        _add2_body,
        grid=(m // bm,),
        in_specs=[spec, spec, pl.BlockSpec((1, ch), lambda i: (0, 0))],
        out_specs=spec,
        out_shape=jax.ShapeDtypeStruct((m, ch), jnp.float32),
    )(a, b, brow)


def _vgather(vec, idx16):
    dnums = lax.GatherDimensionNumbers(
        offset_dims=(), collapsed_slice_dims=(0,), start_index_map=(0,))
    return lax.gather(vec, idx16[:, None], dnums, (1,),
                      mode=lax.GatherScatterMode.PROMISE_IN_BOUNDS)


def _spmm_partials(rows, cols, vals, table, init):
    nnzp = vals.shape[0]
    m, c = table.shape
    nt = nnzp // (NW * CH)
    rpt = 8 * (m // 8 // NS)
    rem = m - NS * rpt

    mesh = plsc.VectorSubcoreMesh(core_axis_name="c", subcore_axis_name="s")

    scratch = [
        pltpu.VMEM((CH, c), jnp.float32),
        pltpu.VMEM((CH,), jnp.int32),
        pltpu.VMEM((CH,), jnp.int32),
        pltpu.VMEM((CH,), jnp.float32),
        pltpu.VMEM_SHARED((m, c), jnp.float32),
        pltpu.SemaphoreType.DMA,
    ]

    @functools.partial(
        pl.kernel,
        out_type=jax.ShapeDtypeStruct((NC, m, c), jnp.float32),
        mesh=mesh,
        scratch_types=scratch,
    )
    def spmm(rows_hbm, cols_hbm, vals_hbm, table_hbm, init_hbm, out_hbm,
             gath, rowv, colv, valv, acc, sem):
        cid = lax.axis_index("c")
        sid = lax.axis_index("s")
        wid = cid * NS + sid

        pltpu.sync_copy(init_hbm.at[cid, pl.ds(sid * rpt, rpt)],
                        acc.at[pl.ds(sid * rpt, rpt)])
        if rem:
            @pl.when(sid == NS - 1)
            def _():
                pltpu.sync_copy(init_hbm.at[cid, pl.ds(NS * rpt, rem)],
                                acc.at[pl.ds(NS * rpt, rem)])
        plsc.subcore_barrier()

        def chunk_body(t, carry):
            e0 = (wid + t * NW) * CH
            pltpu.sync_copy(rows_hbm.at[pl.ds(e0, CH)], rowv)
            pltpu.sync_copy(cols_hbm.at[pl.ds(e0, CH)], colv)
            pltpu.sync_copy(vals_hbm.at[pl.ds(e0, CH)], valv)
            pltpu.async_copy(table_hbm.at[colv], gath, sem).wait()

            def scale_block(eb, cc):
                vblock = valv[pl.ds(eb * LANES, LANES)]
                for l in range(LANES):
                    vv = _vgather(vblock, jnp.full((LANES,), l, jnp.int32))
                    e = eb * LANES + l
                    for j in range(c // LANES):
                        g = gath[e, pl.ds(j * LANES, LANES)]
                        gath[e, pl.ds(j * LANES, LANES)] = g * vv
                return cc

            lax.fori_loop(0, CH // LANES, scale_block, 0)
            pltpu.sync_copy(gath, acc.at[rowv], add=True)
            return carry

        lax.fori_loop(0, nt, chunk_body, 0)
        plsc.subcore_barrier()

        pltpu.sync_copy(acc.at[pl.ds(sid * rpt, rpt)],
                        out_hbm.at[cid, pl.ds(sid * rpt, rpt)])
        if rem:
            @pl.when(sid == NS - 1)
            def _():
                pltpu.sync_copy(acc.at[pl.ds(NS * rpt, rem)],
                                out_hbm.at[cid, pl.ds(NS * rpt, rem)])

    return spmm(rows, cols, vals, table, init)


def kernel(L_indices, L_values, x, theta, bias):
    rows = L_indices[0].astype(jnp.int32)
    cols = L_indices[1].astype(jnp.int32)
    vals = L_values.astype(jnp.float32)

    cout, cin, k = theta.shape
    nnz = vals.shape[0]

    quant = NW * CH
    nnzp = ((nnz + quant - 1) // quant) * quant
    pad = nnzp - nnz
    if pad:
        rows = jnp.concatenate([rows, jnp.zeros((pad,), jnp.int32)])
        cols = jnp.concatenate([cols, jnp.zeros((pad,), jnp.int32)])
        vals = jnp.concatenate([vals, jnp.zeros((pad,), jnp.float32)])

    thetaT = jnp.transpose(theta, (1, 2, 0)).reshape(cin, k * cout)
    biasT = bias[0, :, 0][None, :]
    zrow = jnp.zeros((1, cout), jnp.float32)
    zeros_mc = jnp.zeros((x.shape[2], cout), jnp.float32)

    z0, z1, z2 = _mm3(x[0], thetaT)

    u_p = _spmm_partials(rows, cols, vals, z2, jnp.stack([z1, zeros_mc]))
    u = _add2(u_p[0], u_p[1], zrow)
    y_p = _spmm_partials(rows, cols, vals, u, jnp.stack([z0, zeros_mc]))
    yT = _add2(y_p[0], y_p[1], biasT)
    return yT.T[None]


# speedup vs baseline: 1.0617x; 1.0070x over previous
import functools

import jax
import jax.numpy as jnp
from jax import lax
from jax.experimental import pallas as pl
from jax.experimental.pallas import tpu as pltpu
from jax.experimental.pallas import tpu_sc as plsc

NC = 2
NS = 16
NW = NC * NS
CH = 128
LANES = 16


def _mm_body(x_ref, t_ref, o0_ref, o1_ref, o2_ref):
    y = jnp.dot(x_ref[...], t_ref[...], preferred_element_type=jnp.float32)
    c = o0_ref.shape[1]
    o0_ref[...] = y[:, 0:c]
    o1_ref[...] = y[:, c:2 * c]
    o2_ref[...] = y[:, 2 * c:3 * c]


def _mm3(xT, thetaT, bm=2000):
    m, cin = xT.shape
    ck3 = thetaT.shape[1]
    c = ck3 // 3
    out = jax.ShapeDtypeStruct((m, c), jnp.float32)
    return pl.pallas_call(
        _mm_body,
        grid=(m // bm,),
        in_specs=[
            pl.BlockSpec((bm, cin), lambda i: (i, 0)),
            pl.BlockSpec((cin, ck3), lambda i: (0, 0)),
        ],
        out_specs=[pl.BlockSpec((bm, c), lambda i: (i, 0))] * 3,
        out_shape=[out, out, out],
    )(xT, thetaT)


def _add2_body(a_ref, b_ref, d_ref, o_ref):
    o_ref[...] = a_ref[...] + b_ref[...] + d_ref[...]


def _add2(a, b, brow, bm=2000):
    m, ch = a.shape
    spec = pl.BlockSpec((bm, ch), lambda i: (i, 0))
    return pl.pallas_call(
        _add2_body,
        grid=(m // bm,),
        in_specs=[spec, spec, pl.BlockSpec((1, ch), lambda i: (0, 0))],
        out_specs=spec,
        out_shape=jax.ShapeDtypeStruct((m, ch), jnp.float32),
    )(a, b, brow)


def _vgather(vec, idx16):
    dnums = lax.GatherDimensionNumbers(
        offset_dims=(), collapsed_slice_dims=(0,), start_index_map=(0,))
    return lax.gather(vec, idx16[:, None], dnums, (1,),
                      mode=lax.GatherScatterMode.PROMISE_IN_BOUNDS)


def _spmm_partials(rows, cols, vals, table, init):
    nnzp = vals.shape[0]
    m, c = table.shape
    nt = nnzp // (NW * CH)
    rpt = 8 * (m // 8 // NS)
    rem = m - NS * rpt

    mesh = plsc.VectorSubcoreMesh(core_axis_name="c", subcore_axis_name="s")

    scratch = [
        pltpu.VMEM((CH, c), jnp.float32),
        pltpu.VMEM((CH,), jnp.int32),
        pltpu.VMEM((CH,), jnp.int32),
        pltpu.VMEM((CH,), jnp.float32),
        pltpu.VMEM_SHARED((m, c), jnp.float32),
        pltpu.SemaphoreType.DMA,
    ]

    @functools.partial(
        pl.kernel,
        out_type=jax.ShapeDtypeStruct((NC, m, c), jnp.float32),
        mesh=mesh,
        scratch_types=scratch,
    )
    def spmm(rows_hbm, cols_hbm, vals_hbm, table_hbm, init_hbm, out_hbm,
             gath, rowv, colv, valv, acc, sem):
        cid = lax.axis_index("c")
        sid = lax.axis_index("s")
        wid = cid * NS + sid

        pltpu.sync_copy(init_hbm.at[cid, pl.ds(sid * rpt, rpt)],
                        acc.at[pl.ds(sid * rpt, rpt)])
        if rem:
            @pl.when(sid == NS - 1)
            def _():
                pltpu.sync_copy(init_hbm.at[cid, pl.ds(NS * rpt, rem)],
                                acc.at[pl.ds(NS * rpt, rem)])
        plsc.subcore_barrier()

        def chunk_body(t, carry):
            e0 = (wid + t * NW) * CH
            pltpu.sync_copy(rows_hbm.at[pl.ds(e0, CH)], rowv)
            pltpu.sync_copy(cols_hbm.at[pl.ds(e0, CH)], colv)
            pltpu.sync_copy(vals_hbm.at[pl.ds(e0, CH)], valv)
            pltpu.async_copy(table_hbm.at[colv], gath, sem).wait()

            def scale_block(eb, cc):
                vblock = valv[pl.ds(eb * LANES, LANES)]
                for l in range(LANES):
                    vv = _vgather(vblock, jnp.full((LANES,), l, jnp.int32))
                    e = eb * LANES + l
                    for j in range(c // LANES):
                        g = gath[e, pl.ds(j * LANES, LANES)]
                        gath[e, pl.ds(j * LANES, LANES)] = g * vv
                return cc

            lax.fori_loop(0, CH // LANES, scale_block, 0)
            pltpu.sync_copy(gath, acc.at[rowv], add=True)
            return carry

        lax.fori_loop(0, nt, chunk_body, 0)
        plsc.subcore_barrier()

        pltpu.sync_copy(acc.at[pl.ds(sid * rpt, rpt)],
                        out_hbm.at[cid, pl.ds(sid * rpt, rpt)])
        if rem:
            @pl.when(sid == NS - 1)
            def _():
                pltpu.sync_copy(acc.at[pl.ds(NS * rpt, rem)],
                                out_hbm.at[cid, pl.ds(NS * rpt, rem)])

    return spmm(rows, cols, vals, table, init)


def kernel(L_indices, L_values, x, theta, bias):
    rows = L_indices[0].astype(jnp.int32)
    cols = L_indices[1].astype(jnp.int32)
    vals = L_values.astype(jnp.float32)

    cout, cin, k = theta.shape
    nnz = vals.shape[0]

    quant = NW * CH
    nnzp = ((nnz + quant - 1) // quant) * quant
    pad = nnzp - nnz
    if pad:
        rows = jnp.concatenate([rows, jnp.zeros((pad,), jnp.int32)])
        cols = jnp.concatenate([cols, jnp.zeros((pad,), jnp.int32)])
        vals = jnp.concatenate([vals, jnp.zeros((pad,), jnp.float32)])

    thetaT = jnp.transpose(theta, (1, 2, 0)).reshape(cin, k * cout)
    biasT = bias[0, :, 0][None, :]
    zrow = jnp.zeros((1, cout), jnp.float32)
    zeros_mc = jnp.zeros((x.shape[2], cout), jnp.float32)

    xT = x[0].T
    z0, z1, z2 = _mm3(xT, thetaT)

    u_p = _spmm_partials(rows, cols, vals, z2, jnp.stack([z1, zeros_mc]))
    u = _add2(u_p[0], u_p[1], zrow)
    y_p = _spmm_partials(rows, cols, vals, u, jnp.stack([z0, zeros_mc]))
    yT = _add2(y_p[0], y_p[1], biasT)
    return yT.T[None]


# speedup vs baseline: 1.4034x; 1.3219x over previous
import functools

import jax
import jax.numpy as jnp
from jax import lax
from jax.experimental import pallas as pl
from jax.experimental.pallas import tpu as pltpu
from jax.experimental.pallas import tpu_sc as plsc

NC = 2
NS = 16
NW = NC * NS
CH = 128
LANES = 16


def _mm_body(x_ref, t_ref, o0_ref, o1_ref, o2_ref):
    y = jnp.dot(x_ref[...], t_ref[...], preferred_element_type=jnp.float32)
    c = o0_ref.shape[1]
    o0_ref[...] = y[:, 0:c]
    o1_ref[...] = y[:, c:2 * c]
    o2_ref[...] = y[:, 2 * c:3 * c]


def _mm3(xT, thetaT, bm=2000):
    m, cin = xT.shape
    ck3 = thetaT.shape[1]
    c = ck3 // 3
    out = jax.ShapeDtypeStruct((m, c), jnp.float32)
    return pl.pallas_call(
        _mm_body,
        grid=(m // bm,),
        in_specs=[
            pl.BlockSpec((bm, cin), lambda i: (i, 0)),
            pl.BlockSpec((cin, ck3), lambda i: (0, 0)),
        ],
        out_specs=[pl.BlockSpec((bm, c), lambda i: (i, 0))] * 3,
        out_shape=[out, out, out],
    )(xT, thetaT)


def _add2_body(a_ref, b_ref, d_ref, o_ref):
    o_ref[...] = a_ref[...] + b_ref[...] + d_ref[...]


def _add2(a, b, brow, bm=2000):
    m, ch = a.shape
    spec = pl.BlockSpec((bm, ch), lambda i: (i, 0))
    return pl.pallas_call(
        _add2_body,
        grid=(m // bm,),
        in_specs=[spec, spec, pl.BlockSpec((1, ch), lambda i: (0, 0))],
        out_specs=spec,
        out_shape=jax.ShapeDtypeStruct((m, ch), jnp.float32),
    )(a, b, brow)


def _vgather(vec, idx16):
    dnums = lax.GatherDimensionNumbers(
        offset_dims=(), collapsed_slice_dims=(0,), start_index_map=(0,))
    return lax.gather(vec, idx16[:, None], dnums, (1,),
                      mode=lax.GatherScatterMode.PROMISE_IN_BOUNDS)


def _spmm_partials(rows, cols, vals, table, init):
    nnzp = vals.shape[0]
    m, c = table.shape
    nt = nnzp // (NW * CH)
    rpt = 8 * (m // 8 // NS)
    rem = m - NS * rpt

    mesh = plsc.VectorSubcoreMesh(core_axis_name="c", subcore_axis_name="s")

    scratch = [
        pltpu.VMEM((CH, c), jnp.float32),
        pltpu.VMEM((CH,), jnp.int32),
        pltpu.VMEM((CH,), jnp.int32),
        pltpu.VMEM((CH,), jnp.float32),
        pltpu.VMEM_SHARED((m, c), jnp.float32),
        pltpu.SemaphoreType.DMA,
    ]

    @functools.partial(
        pl.kernel,
        out_type=jax.ShapeDtypeStruct((NC, m, c), jnp.float32),
        mesh=mesh,
        scratch_types=scratch,
    )
    def spmm(rows_hbm, cols_hbm, vals_hbm, table_hbm, init_hbm, out_hbm,
             gath, rowv, colv, valv, acc, sem):
        cid = lax.axis_index("c")
        sid = lax.axis_index("s")
        wid = cid * NS + sid

        pltpu.sync_copy(init_hbm.at[cid, pl.ds(sid * rpt, rpt)],
                        acc.at[pl.ds(sid * rpt, rpt)])
        if rem:
            @pl.when(sid == NS - 1)
            def _():
                pltpu.sync_copy(init_hbm.at[cid, pl.ds(NS * rpt, rem)],
                                acc.at[pl.ds(NS * rpt, rem)])
        plsc.subcore_barrier()

        def chunk_body(t, carry):
            e0 = (wid + t * NW) * CH
            pltpu.sync_copy(rows_hbm.at[pl.ds(e0, CH)], rowv)
            pltpu.sync_copy(cols_hbm.at[pl.ds(e0, CH)], colv)
            pltpu.sync_copy(vals_hbm.at[pl.ds(e0, CH)], valv)
            pltpu.async_copy(table_hbm.at[colv], gath, sem).wait()

            def scale_block(eb, cc):
                vblock = valv[pl.ds(eb * LANES, LANES)]
                for l in range(LANES):
                    vv = _vgather(vblock, jnp.full((LANES,), l, jnp.int32))
                    e = eb * LANES + l
                    for j in range(c // LANES):
                        g = gath[e, pl.ds(j * LANES, LANES)]
                        gath[e, pl.ds(j * LANES, LANES)] = g * vv
                return cc

            lax.fori_loop(0, CH // LANES, scale_block, 0)
            pltpu.sync_copy(gath, acc.at[rowv], add=True)
            return carry

        lax.fori_loop(0, nt, chunk_body, 0)
        plsc.subcore_barrier()

        pltpu.sync_copy(acc.at[pl.ds(sid * rpt, rpt)],
                        out_hbm.at[cid, pl.ds(sid * rpt, rpt)])
        if rem:
            @pl.when(sid == NS - 1)
            def _():
                pltpu.sync_copy(acc.at[pl.ds(NS * rpt, rem)],
                                out_hbm.at[cid, pl.ds(NS * rpt, rem)])

    return spmm(rows, cols, vals, table, init)


def kernel(L_indices, L_values, x, theta, bias):
    rows = L_indices[0].astype(jnp.int32)
    cols = L_indices[1].astype(jnp.int32)
    vals = L_values.astype(jnp.float32)

    cout, cin, k = theta.shape
    nnz = vals.shape[0]

    quant = NW * CH
    nnzp = ((nnz + quant - 1) // quant) * quant
    pad = nnzp - nnz
    m = x.shape[2]
    if pad:
        spread = (jnp.arange(pad, dtype=jnp.int32) * 8) % m
        rows = jnp.concatenate([rows, spread])
        cols = jnp.concatenate([cols, spread])
        vals = jnp.concatenate([vals, jnp.zeros((pad,), jnp.float32)])

    thetaT = jnp.transpose(theta, (1, 2, 0)).reshape(cin, k * cout)
    biasT = bias[0, :, 0][None, :]
    zrow = jnp.zeros((1, cout), jnp.float32)
    zeros_mc = jnp.zeros((x.shape[2], cout), jnp.float32)

    xT = x[0].T
    z0, z1, z2 = _mm3(xT, thetaT)

    u_p = _spmm_partials(rows, cols, vals, z2, jnp.stack([z1, zeros_mc]))
    u = _add2(u_p[0], u_p[1], zrow)
    y_p = _spmm_partials(rows, cols, vals, u, jnp.stack([z0, zeros_mc]))
    yT = _add2(y_p[0], y_p[1], biasT)
    return yT.T[None]
